# baseline (speedup 1.0000x reference)
import functools
import jax
import jax.numpy as jnp
from jax import lax
from jax.experimental import pallas as pl
from jax.experimental.pallas import tpu as pltpu
from jax.experimental.pallas import tpu_sc as plsc

D = 256
NTOPIC = 20
NLAYERS = 3
NCHUNK = 8
CW = 32

NODE_TYPES = ("gene", "peak", "protein")
EDGE_DEFS = (("g2p", "gene", "peak"), ("g2pr", "gene", "protein"),
             ("p2pr", "peak", "protein"), ("pr2pr", "protein", "protein"))
IN_EDGES = {"gene": (), "peak": ("g2p",), "protein": ("g2pr", "p2pr", "pr2pr")}
OUT_EDGES = {"gene": ("g2p", "g2pr"), "peak": ("p2pr",), "protein": ("pr2pr",)}


def _row_block(n):
    if n % 1000 == 0:
        return 1000
    return n


def _proj_kernel(n_edge, x_ref, w_ref, self_ref, *edge_refs):
    acc = jnp.dot(x_ref[...], w_ref[...], preferred_element_type=jnp.float32)
    self_ref[...] = acc[:, :D]
    for j in range(n_edge):
        base = D * (1 + j)
        for c in range(NCHUNK):
            edge_refs[j][c] = acc[:, base + c * CW: base + (c + 1) * CW]


def _project(x, w_self, w_edges):
    n = x.shape[0]
    bn = _row_block(n)
    w = jnp.concatenate([w_self] + list(w_edges), axis=1)
    kout = w.shape[1]
    n_edge = len(w_edges)
    out_shapes = [jax.ShapeDtypeStruct((n, D), jnp.float32)] + [
        jax.ShapeDtypeStruct((NCHUNK, n, CW), jnp.float32) for _ in range(n_edge)
    ]
    out_specs = [pl.BlockSpec((bn, D), lambda i: (i, 0))] + [
        pl.BlockSpec((NCHUNK, bn, CW), lambda i: (0, i, 0)) for _ in range(n_edge)
    ]
    return pl.pallas_call(
        functools.partial(_proj_kernel, n_edge),
        grid=(n // bn,),
        in_specs=[
            pl.BlockSpec((bn, D), lambda i: (i, 0)),
            pl.BlockSpec((D, kout), lambda i: (0, 0)),
        ],
        out_specs=out_specs,
        out_shape=out_shapes,
    )(x, w)


def _combine_kernel(n_edge, self_ref, h_ref, *rest):
    out_ref = rest[-1]
    total = self_ref[...]
    for j in range(n_edge):
        agg_ref = rest[2 * j]
        cnt_ref = rest[2 * j + 1]
        agg = jnp.concatenate([agg_ref[c] for c in range(NCHUNK)], axis=1)
        cnt = jnp.maximum(cnt_ref[:, 0:1], 1.0)
        total = total + agg / cnt
    out_ref[...] = jnp.maximum(total, 0.0) + h_ref[...]


def _combine(selfp, h, aggs, cnts):
    n = h.shape[0]
    bn = _row_block(n)
    n_edge = len(aggs)
    in_specs = [pl.BlockSpec((bn, D), lambda i: (i, 0)),
                pl.BlockSpec((bn, D), lambda i: (i, 0))]
    args = [selfp, h]
    for agg, cnt in zip(aggs, cnts):
        in_specs.append(pl.BlockSpec((NCHUNK, bn, CW), lambda i: (0, i, 0)))
        in_specs.append(pl.BlockSpec((bn, CW), lambda i: (i, 0)))
        args.append(agg)
        args.append(cnt)
    return pl.pallas_call(
        functools.partial(_combine_kernel, n_edge),
        grid=(n // bn,),
        in_specs=in_specs,
        out_specs=pl.BlockSpec((bn, D), lambda i: (i, 0)),
        out_shape=jax.ShapeDtypeStruct((n, D), jnp.float32),
    )(*args)


def _encoder_kernel(x_ref, w1_ref, b1_ref, w2_ref, b2_ref, w3_ref, b3_ref,
                    out_ref):
    h1 = jnp.maximum(
        jnp.dot(x_ref[...], w1_ref[...], preferred_element_type=jnp.float32)
        + b1_ref[...], 0.0)
    h2 = jnp.maximum(
        jnp.dot(h1, w2_ref[...], preferred_element_type=jnp.float32)
        + b2_ref[...], 0.0)
    mu = jnp.dot(h2, w3_ref[...], preferred_element_type=jnp.float32) + b3_ref[...]
    mu = mu - jnp.max(mu, axis=-1, keepdims=True)
    e = jnp.exp(mu)
    out_ref[...] = e / jnp.sum(e, axis=-1, keepdims=True)


def _encode(x, w1, b1, w2, b2, w3, b3):
    n = x.shape[0]
    bn = _row_block(n)
    h1 = w1.shape[1]
    h2 = w2.shape[1]
    return pl.pallas_call(
        _encoder_kernel,
        grid=(n // bn,),
        in_specs=[
            pl.BlockSpec((bn, D), lambda i: (i, 0)),
            pl.BlockSpec((D, h1), lambda i: (0, 0)),
            pl.BlockSpec((1, h1), lambda i: (0, 0)),
            pl.BlockSpec((h1, h2), lambda i: (0, 0)),
            pl.BlockSpec((1, h2), lambda i: (0, 0)),
            pl.BlockSpec((h2, NTOPIC), lambda i: (0, 0)),
            pl.BlockSpec((1, NTOPIC), lambda i: (0, 0)),
        ],
        out_specs=pl.BlockSpec((bn, NTOPIC), lambda i: (i, 0)),
        out_shape=jax.ShapeDtypeStruct((n, NTOPIC), jnp.float32),
    )(x, w1, b1.reshape(1, -1), w2, b2.reshape(1, -1), w3, b3.reshape(1, -1))


SC_NCORE = 2
SC_NSUB = 16
EB = 128
K_GRP = 2
NBUF = 2
ZROWS = 64

_SC_ETS = (
    ("g2p", 10000, 50000, 50048, 51200, 163840),
    ("g2pr", 10000, 2000, 2048, 2048, 40960),
    ("p2pr", 50000, 2000, 2048, 2048, 81920),
    ("pr2pr", 2000, 2000, 2048, 2048, 24576),
)
ACC_ROWS = 51200


def _sc_agg_body(with_counts, *refs):
    n_out = 8 if with_counts else 4
    ins = refs[:12]
    outs = refs[12:12 + n_out]
    if with_counts:
        acc, idx_s, idx_d, rows, zbuf, ones, gsem0, gsem1, ssem = \
            refs[12 + n_out:]
    else:
        acc, idx_s, idx_d, rows, zbuf, gsem0, gsem1, ssem = refs[12 + n_out:]
    gsems = (gsem0, gsem1)
    cid = lax.axis_index("c")
    sid = lax.axis_index("s")

    def initz(i, carry):
        for j in range(CW // 16):
            zbuf[i, pl.ds(16 * j, 16)] = jnp.zeros((16,), jnp.float32)
        return carry

    lax.fori_loop(0, ZROWS, initz, 0)
    if with_counts:
        def inito(i, carry):
            for j in range(CW // 16):
                ones[i, pl.ds(16 * j, 16)] = jnp.ones((16,), jnp.float32)
            return carry

        lax.fori_loop(0, EB, inito, 0)

    for ei, (name, n_src, n_dst, nout, npad, epad) in enumerate(_SC_ETS):
        proj = ins[3 * ei]
        src2 = ins[3 * ei + 1]
        dst2 = ins[3 * ei + 2]
        out = outs[ei]
        ept = epad // SC_NSUB
        nb = ept // EB
        ngrp = nb // K_GRP
        rpt_zero = npad // SC_NSUB
        nzblk = rpt_zero // ZROWS
        wrt = nout // SC_NSUB

        dummy_rows = proj.at[0].at[pl.ds(0, EB)]

        def zero_acc():
            def zb(i, carry):
                d0 = pltpu.async_copy(
                    zbuf,
                    acc.at[pl.ds(sid * rpt_zero + 2 * i * ZROWS, ZROWS)],
                    gsem0)
                d1 = pltpu.async_copy(
                    zbuf,
                    acc.at[pl.ds(sid * rpt_zero + (2 * i + 1) * ZROWS, ZROWS)],
                    gsem1)
                d0.wait()
                d1.wait()
                return carry

            lax.fori_loop(0, nzblk // 2, zb, 0)

        def load_idx(g, slot):
            r0 = sid * nb + g * K_GRP
            pltpu.sync_copy(src2.at[pl.ds(r0, K_GRP)],
                            idx_s.at[pl.ds(slot * K_GRP, K_GRP)])
            pltpu.sync_copy(dst2.at[pl.ds(r0, K_GRP)],
                            idx_d.at[pl.ds(slot * K_GRP, K_GRP)])

        for c_l in range(NCHUNK // SC_NCORE):
            chunk = cid * (NCHUNK // SC_NCORE) + c_l
            zero_acc()
            plsc.subcore_barrier()
            ptab = proj.at[chunk]

            def fire(bank, g):
                load_idx(g, bank)
                return [
                    pltpu.async_copy(ptab.at[idx_s.at[bank * K_GRP + k]],
                                     rows.at[bank * K_GRP + k], gsems[bank])
                    for k in range(K_GRP)
                ]

            def scatter(bank):
                sds = [
                    pltpu.async_copy(rows.at[bank * K_GRP + k],
                                     acc.at[idx_d.at[bank * K_GRP + k]],
                                     ssem, add=True)
                    for k in range(K_GRP)
                ]
                for d in sds:
                    d.wait()

            def grp(i, carry):
                g0 = 2 * i
                da = fire(0, g0)
                db = fire(1, g0 + 1)
                for d in da:
                    d.wait()
                scatter(0)
                for d in db:
                    d.wait()
                scatter(1)
                return carry

            lax.fori_loop(0, ngrp // 2, grp, 0)
            plsc.subcore_barrier()
            pltpu.sync_copy(acc.at[pl.ds(sid * wrt, wrt)],
                            out.at[chunk].at[pl.ds(sid * wrt, wrt)])
            plsc.subcore_barrier()

        if with_counts:
            cnt_out = outs[4 + ei]
            owner = 0 if ei < 2 else 1

            @pl.when(cid == owner)
            def _():
                zero_acc()
                plsc.subcore_barrier()

                def cscatter(bank):
                    sds = [
                        pltpu.async_copy(ones,
                                         acc.at[idx_d.at[bank * K_GRP + k]],
                                         ssem, add=True)
                        for k in range(K_GRP)
                    ]
                    for d in sds:
                        d.wait()

                def cgrp(i, carry):
                    g0 = 2 * i
                    load_idx(g0, 0)
                    load_idx(g0 + 1, 1)
                    cscatter(0)
                    cscatter(1)
                    return carry

                lax.fori_loop(0, ngrp // 2, cgrp, 0)
                plsc.subcore_barrier()
                pltpu.sync_copy(acc.at[pl.ds(sid * wrt, wrt)],
                                cnt_out.at[pl.ds(sid * wrt, wrt)])
                plsc.subcore_barrier()


def _sc_aggregate(projs, srcs2, dsts2, with_counts):
    out_type = [jax.ShapeDtypeStruct((NCHUNK, et[3], CW), jnp.float32)
                for et in _SC_ETS]
    if with_counts:
        out_type += [jax.ShapeDtypeStruct((et[3], CW), jnp.float32)
                     for et in _SC_ETS]
    scratch = [
        pltpu.VMEM_SHARED((ACC_ROWS, CW), jnp.float32),
        pltpu.VMEM((NBUF * K_GRP, EB), jnp.int32),
        pltpu.VMEM((NBUF * K_GRP, EB), jnp.int32),
        pltpu.VMEM((NBUF * K_GRP, EB, CW), jnp.float32),
        pltpu.VMEM((ZROWS, CW), jnp.float32),
    ]
    if with_counts:
        scratch.append(pltpu.VMEM((EB, CW), jnp.float32))
    scratch += [pltpu.SemaphoreType.DMA, pltpu.SemaphoreType.DMA,
                pltpu.SemaphoreType.DMA]
    mesh = plsc.VectorSubcoreMesh(core_axis_name="c", subcore_axis_name="s")
    fn = pl.kernel(
        functools.partial(_sc_agg_body, with_counts),
        out_type=out_type,
        mesh=mesh,
        scratch_types=scratch,
        compiler_params=pltpu.CompilerParams(use_tc_tiling_on_sc=False),
    )
    args = []
    for et in _SC_ETS:
        args += [projs[et[0]], srcs2[et[0]], dsts2[et[0]]]
    res = fn(*args)
    aggs = {et[0]: res[i] for i, et in enumerate(_SC_ETS)}
    cnts = None
    if with_counts:
        cnts = {et[0]: res[4 + i] for i, et in enumerate(_SC_ETS)}
    return aggs, cnts


def _pad_edges(edge, n_src, n_dst, npad, epad):
    e = edge.shape[1]
    extra = epad - e
    fill_src = jnp.arange(extra, dtype=jnp.int32) % n_src
    fill_dst = n_dst + jnp.arange(extra, dtype=jnp.int32) % (npad - n_dst)
    src = jnp.concatenate([edge[0], fill_src]).reshape(epad // EB, EB)
    dst = jnp.concatenate([edge[1], fill_dst]).reshape(epad // EB, EB)
    return src, dst


def kernel(gene_x, peak_x, protein_x, params, edge_g2p, edge_g2pr, edge_p2pr,
           edge_pr2pr):
    h = {"gene": gene_x, "peak": peak_x, "protein": protein_x}
    edges = {"g2p": edge_g2p, "g2pr": edge_g2pr, "p2pr": edge_p2pr,
             "pr2pr": edge_pr2pr}

    srcs2 = {}
    dsts2 = {}
    for name, n_src, n_dst, nout, npad, epad in _SC_ETS:
        srcs2[name], dsts2[name] = _pad_edges(edges[name], n_src, n_dst, npad,
                                              epad)

    cnts = None
    for l in range(NLAYERS):
        projs = {}
        selfs = {}
        for nt in NODE_TYPES:
            w_edges = [params["W_%s_%d" % (name, l)] for name in OUT_EDGES[nt]]
            outs = _project(h[nt], params["Wself_%s_%d" % (nt, l)], w_edges)
            selfs[nt] = outs[0]
            for j, name in enumerate(OUT_EDGES[nt]):
                projs[name] = outs[1 + j]

        aggs, new_cnts = _sc_aggregate(projs, srcs2, dsts2, cnts is None)
        if new_cnts is not None:
            cnts = new_cnts

        new_h = {}
        for nt in NODE_TYPES:
            names = IN_EDGES[nt]
            new_h[nt] = _combine(selfs[nt], h[nt],
                                 [aggs[m] for m in names],
                                 [cnts[m] for m in names])
        h = new_h

    outs = []
    for nt in NODE_TYPES:
        p = params
        outs.append(_encode(h[nt], p["enc_%s_W1" % nt], p["enc_%s_b1" % nt],
                            p["enc_%s_W2" % nt], p["enc_%s_b2" % nt],
                            p["enc_%s_W3" % nt], p["enc_%s_b3" % nt]))
    return tuple(outs)


# speedup vs baseline: 1.5204x; 1.5204x over previous
import functools
import jax
import jax.numpy as jnp
from jax import lax
from jax.experimental import pallas as pl
from jax.experimental.pallas import tpu as pltpu
from jax.experimental.pallas import tpu_sc as plsc

D = 256
NTOPIC = 20
NLAYERS = 3
NCHUNK = 8
CW = 32

NODE_TYPES = ("gene", "peak", "protein")
EDGE_DEFS = (("g2p", "gene", "peak"), ("g2pr", "gene", "protein"),
             ("p2pr", "peak", "protein"), ("pr2pr", "protein", "protein"))
IN_EDGES = {"gene": (), "peak": ("g2p",), "protein": ("g2pr", "p2pr", "pr2pr")}
OUT_EDGES = {"gene": ("g2p", "g2pr"), "peak": ("p2pr",), "protein": ("pr2pr",)}


def _row_block(n):
    if n % 1000 == 0:
        return 1000
    return n


def _proj_kernel(chunked, x_ref, w_ref, self_ref, *edge_refs):
    acc = jnp.dot(x_ref[...], w_ref[...], preferred_element_type=jnp.float32)
    self_ref[...] = acc[:, :D]
    for j, ch in enumerate(chunked):
        base = D * (1 + j)
        if ch:
            for c in range(NCHUNK):
                edge_refs[j][c] = acc[:, base + c * CW: base + (c + 1) * CW]
        else:
            edge_refs[j][...] = acc[:, base: base + D]


def _project(x, w_self, w_edges, chunked):
    n = x.shape[0]
    bn = _row_block(n)
    w = jnp.concatenate([w_self] + list(w_edges), axis=1)
    kout = w.shape[1]
    out_shapes = [jax.ShapeDtypeStruct((n, D), jnp.float32)]
    out_specs = [pl.BlockSpec((bn, D), lambda i: (i, 0))]
    for ch in chunked:
        if ch:
            out_shapes.append(jax.ShapeDtypeStruct((NCHUNK, n, CW), jnp.float32))
            out_specs.append(pl.BlockSpec((NCHUNK, bn, CW), lambda i: (0, i, 0)))
        else:
            out_shapes.append(jax.ShapeDtypeStruct((n, D), jnp.float32))
            out_specs.append(pl.BlockSpec((bn, D), lambda i: (i, 0)))
    return pl.pallas_call(
        functools.partial(_proj_kernel, tuple(chunked)),
        grid=(n // bn,),
        in_specs=[
            pl.BlockSpec((bn, D), lambda i: (i, 0)),
            pl.BlockSpec((D, kout), lambda i: (0, 0)),
        ],
        out_specs=out_specs,
        out_shape=out_shapes,
    )(x, w)


def _combine_kernel(chunked, self_ref, h_ref, *rest):
    out_ref = rest[-1]
    total = self_ref[...]
    for j, ch in enumerate(chunked):
        agg_ref = rest[2 * j]
        cnt_ref = rest[2 * j + 1]
        if ch:
            agg = jnp.concatenate([agg_ref[c] for c in range(NCHUNK)], axis=1)
        else:
            agg = agg_ref[0] + agg_ref[1]
        cnt = jnp.maximum(cnt_ref[:, 0:1], 1.0)
        total = total + agg / cnt
    out_ref[...] = jnp.maximum(total, 0.0) + h_ref[...]


def _combine(selfp, h, aggs, cnts, chunked):
    n = h.shape[0]
    bn = _row_block(n)
    in_specs = [pl.BlockSpec((bn, D), lambda i: (i, 0)),
                pl.BlockSpec((bn, D), lambda i: (i, 0))]
    args = [selfp, h]
    for agg, cnt, ch in zip(aggs, cnts, chunked):
        if ch:
            in_specs.append(pl.BlockSpec((NCHUNK, bn, CW), lambda i: (0, i, 0)))
        else:
            in_specs.append(pl.BlockSpec((2, bn, D), lambda i: (0, i, 0)))
        in_specs.append(pl.BlockSpec((bn, CW), lambda i: (i, 0)))
        args.append(agg)
        args.append(cnt)
    return pl.pallas_call(
        functools.partial(_combine_kernel, tuple(chunked)),
        grid=(n // bn,),
        in_specs=in_specs,
        out_specs=pl.BlockSpec((bn, D), lambda i: (i, 0)),
        out_shape=jax.ShapeDtypeStruct((n, D), jnp.float32),
    )(*args)


def _encoder_kernel(x_ref, w1_ref, b1_ref, w2_ref, b2_ref, w3_ref, b3_ref,
                    out_ref):
    h1 = jnp.maximum(
        jnp.dot(x_ref[...], w1_ref[...], preferred_element_type=jnp.float32)
        + b1_ref[...], 0.0)
    h2 = jnp.maximum(
        jnp.dot(h1, w2_ref[...], preferred_element_type=jnp.float32)
        + b2_ref[...], 0.0)
    mu = jnp.dot(h2, w3_ref[...], preferred_element_type=jnp.float32) + b3_ref[...]
    mu = mu - jnp.max(mu, axis=-1, keepdims=True)
    e = jnp.exp(mu)
    out_ref[...] = e / jnp.sum(e, axis=-1, keepdims=True)


def _encode(x, w1, b1, w2, b2, w3, b3):
    n = x.shape[0]
    bn = _row_block(n)
    h1 = w1.shape[1]
    h2 = w2.shape[1]
    return pl.pallas_call(
        _encoder_kernel,
        grid=(n // bn,),
        in_specs=[
            pl.BlockSpec((bn, D), lambda i: (i, 0)),
            pl.BlockSpec((D, h1), lambda i: (0, 0)),
            pl.BlockSpec((1, h1), lambda i: (0, 0)),
            pl.BlockSpec((h1, h2), lambda i: (0, 0)),
            pl.BlockSpec((1, h2), lambda i: (0, 0)),
            pl.BlockSpec((h2, NTOPIC), lambda i: (0, 0)),
            pl.BlockSpec((1, NTOPIC), lambda i: (0, 0)),
        ],
        out_specs=pl.BlockSpec((bn, NTOPIC), lambda i: (i, 0)),
        out_shape=jax.ShapeDtypeStruct((n, NTOPIC), jnp.float32),
    )(x, w1, b1.reshape(1, -1), w2, b2.reshape(1, -1), w3, b3.reshape(1, -1))


SC_NCORE = 2
SC_NSUB = 16
EB = 128
K_GRP = 2
NBUF = 2
ZROWS = 64

G2P_NSRC = 10000
G2P_NDST = 50000
G2P_NOUT = 50048
G2P_NPAD = 51200
G2P_EPAD = 163840
PR_NOUT = 2048
PR_EPAD = {"g2pr": 40960, "p2pr": 81920, "pr2pr": 24576}
_PAD_INFO = (
    ("g2p", G2P_NSRC, G2P_NDST, G2P_NPAD, G2P_EPAD),
    ("g2pr", 10000, 2000, PR_NOUT, PR_EPAD["g2pr"]),
    ("p2pr", 50000, 2000, PR_NOUT, PR_EPAD["p2pr"]),
    ("pr2pr", 2000, 2000, PR_NOUT, PR_EPAD["pr2pr"]),
)


def _init_const(buf, nrows, width, val):
    def body(i, carry):
        for j in range(width // 16):
            buf[i, pl.ds(16 * j, 16)] = jnp.full((16,), val, jnp.float32)
        return carry

    lax.fori_loop(0, nrows, body, 0)


def _sc_g2p_body(with_counts, *refs):
    if with_counts:
        (proj, src2, dst2, dstc1, dstc2, dstc3,
         agg_out, cnt1, cnt2, cnt3, cnt4,
         acc, idx_s, idx_d, rows, zbuf, ones, gsem0, gsem1, ssem) = refs
    else:
        (proj, src2, dst2, agg_out,
         acc, idx_s, idx_d, rows, zbuf, gsem0, gsem1, ssem) = refs
    gsems = (gsem0, gsem1)
    cid = lax.axis_index("c")
    sid = lax.axis_index("s")

    _init_const(zbuf, ZROWS, CW, 0.0)
    if with_counts:
        _init_const(ones, EB, CW, 1.0)

    def zero_acc(npad):
        rpt = npad // SC_NSUB

        def zb(i, carry):
            d0 = pltpu.async_copy(
                zbuf, acc.at[pl.ds(sid * rpt + 2 * i * ZROWS, ZROWS)], gsem0)
            d1 = pltpu.async_copy(
                zbuf, acc.at[pl.ds(sid * rpt + (2 * i + 1) * ZROWS, ZROWS)],
                gsem1)
            d0.wait()
            d1.wait()
            return carry

        lax.fori_loop(0, rpt // ZROWS // 2, zb, 0)

    nb = G2P_EPAD // SC_NSUB // EB
    ngrp = nb // K_GRP

    def load_idx(g, slot):
        r0 = sid * nb + g * K_GRP
        pltpu.sync_copy(src2.at[pl.ds(r0, K_GRP)],
                        idx_s.at[pl.ds(slot * K_GRP, K_GRP)])
        pltpu.sync_copy(dst2.at[pl.ds(r0, K_GRP)],
                        idx_d.at[pl.ds(slot * K_GRP, K_GRP)])

    wrt = G2P_NOUT // SC_NSUB
    for c_l in range(NCHUNK // SC_NCORE):
        chunk = cid * (NCHUNK // SC_NCORE) + c_l
        zero_acc(G2P_NPAD)
        plsc.subcore_barrier()
        ptab = proj.at[chunk]

        def fire(bank, g):
            load_idx(g, bank)
            return [
                pltpu.async_copy(ptab.at[idx_s.at[bank * K_GRP + k]],
                                 rows.at[bank * K_GRP + k], gsems[bank])
                for k in range(K_GRP)
            ]

        def scatter(bank):
            sds = [
                pltpu.async_copy(rows.at[bank * K_GRP + k],
                                 acc.at[idx_d.at[bank * K_GRP + k]],
                                 ssem, add=True)
                for k in range(K_GRP)
            ]
            for d in sds:
                d.wait()

        def grp(i, carry):
            g0 = 2 * i
            da = fire(0, g0)
            db = fire(1, g0 + 1)
            for d in da:
                d.wait()
            scatter(0)
            for d in db:
                d.wait()
            scatter(1)
            return carry

        lax.fori_loop(0, ngrp // 2, grp, 0)
        plsc.subcore_barrier()
        pltpu.sync_copy(acc.at[pl.ds(sid * wrt, wrt)],
                        agg_out.at[chunk].at[pl.ds(sid * wrt, wrt)])
        plsc.subcore_barrier()

    if with_counts:
        cdefs = (
            (dst2, G2P_EPAD, 0, cnt1, G2P_NPAD, G2P_NOUT),
            (dstc1, PR_EPAD["g2pr"], 1, cnt2, PR_NOUT, PR_NOUT),
            (dstc2, PR_EPAD["p2pr"], 1, cnt3, PR_NOUT, PR_NOUT),
            (dstc3, PR_EPAD["pr2pr"], 1, cnt4, PR_NOUT, PR_NOUT),
        )
        for dref, epad, owner, cout, npad, nout in cdefs:
            cnb = epad // SC_NSUB // EB
            cngrp = cnb // K_GRP
            cwrt = nout // SC_NSUB

            @pl.when(cid == owner)
            def _():
                zero_acc(npad)
                plsc.subcore_barrier()

                def load_dst(g, slot):
                    r0 = sid * cnb + g * K_GRP
                    pltpu.sync_copy(dref.at[pl.ds(r0, K_GRP)],
                                    idx_d.at[pl.ds(slot * K_GRP, K_GRP)])

                def cscatter(bank):
                    sds = [
                        pltpu.async_copy(ones,
                                         acc.at[idx_d.at[bank * K_GRP + k]],
                                         ssem, add=True)
                        for k in range(K_GRP)
                    ]
                    for d in sds:
                        d.wait()

                def cgrp(i, carry):
                    g0 = 2 * i
                    load_dst(g0, 0)
                    load_dst(g0 + 1, 1)
                    cscatter(0)
                    cscatter(1)
                    return carry

                lax.fori_loop(0, cngrp // 2, cgrp, 0)
                plsc.subcore_barrier()
                pltpu.sync_copy(acc.at[pl.ds(sid * cwrt, cwrt)],
                                cout.at[pl.ds(sid * cwrt, cwrt)])
                plsc.subcore_barrier()


def _sc_g2p(proj, src2, dst2, cdsts):
    with_counts = cdsts is not None
    out_type = [jax.ShapeDtypeStruct((NCHUNK, G2P_NOUT, CW), jnp.float32)]
    if with_counts:
        out_type += [jax.ShapeDtypeStruct((G2P_NOUT, CW), jnp.float32)] + [
            jax.ShapeDtypeStruct((PR_NOUT, CW), jnp.float32) for _ in range(3)]
    scratch = [
        pltpu.VMEM_SHARED((G2P_NPAD, CW), jnp.float32),
        pltpu.VMEM((NBUF * K_GRP, EB), jnp.int32),
        pltpu.VMEM((NBUF * K_GRP, EB), jnp.int32),
        pltpu.VMEM((NBUF * K_GRP, EB, CW), jnp.float32),
        pltpu.VMEM((ZROWS, CW), jnp.float32),
    ]
    if with_counts:
        scratch.append(pltpu.VMEM((EB, CW), jnp.float32))
    scratch += [pltpu.SemaphoreType.DMA] * 3
    mesh = plsc.VectorSubcoreMesh(core_axis_name="c", subcore_axis_name="s")
    fn = pl.kernel(
        functools.partial(_sc_g2p_body, with_counts),
        out_type=out_type,
        mesh=mesh,
        scratch_types=scratch,
        compiler_params=pltpu.CompilerParams(use_tc_tiling_on_sc=False),
    )
    args = [proj, src2, dst2] + (list(cdsts) if with_counts else [])
    res = fn(*args)
    if with_counts:
        return res[0], list(res[1:5])
    return res[0], None


_PR_ETS = ("g2pr", "p2pr", "pr2pr")


def _sc_prot_body(*refs):
    (p1, s1, d1, p2, s2, d2, p3, s3, d3, o1, o2, o3,
     acc, idx_s, idx_d, rows, zbuf, gsem0, gsem1, ssem) = refs
    gsems = (gsem0, gsem1)
    cid = lax.axis_index("c")
    sid = lax.axis_index("s")
    wid = cid * SC_NSUB + sid

    _init_const(zbuf, ZROWS, D, 0.0)

    rpt = PR_NOUT // SC_NSUB

    def zero_acc():
        d0 = pltpu.async_copy(zbuf, acc.at[pl.ds(sid * rpt, ZROWS)], gsem0)
        d1 = pltpu.async_copy(zbuf, acc.at[pl.ds(sid * rpt + ZROWS, ZROWS)],
                              gsem1)
        d0.wait()
        d1.wait()

    wrt = PR_NOUT // SC_NSUB
    for proj, src2, dst2, out, name in ((p1, s1, d1, o1, "g2pr"),
                                        (p2, s2, d2, o2, "p2pr"),
                                        (p3, s3, d3, o3, "pr2pr")):
        epad = PR_EPAD[name]
        nb = epad // (2 * SC_NSUB) // EB

        def load_idx(g, slot):
            r0 = wid * nb + g
            pltpu.sync_copy(src2.at[pl.ds(r0, 1)], idx_s.at[pl.ds(slot, 1)])
            pltpu.sync_copy(dst2.at[pl.ds(r0, 1)], idx_d.at[pl.ds(slot, 1)])

        zero_acc()
        plsc.subcore_barrier()

        def fire(bank, g):
            load_idx(g, bank)
            return pltpu.async_copy(proj.at[idx_s.at[bank]], rows.at[bank],
                                    gsems[bank])

        def scatter(bank):
            pltpu.async_copy(rows.at[bank], acc.at[idx_d.at[bank]], ssem,
                             add=True).wait()

        def grp(i, carry):
            g0 = 2 * i
            da = fire(0, g0)
            db = fire(1, g0 + 1)
            da.wait()
            scatter(0)
            db.wait()
            scatter(1)
            return carry

        lax.fori_loop(0, nb // 2, grp, 0)
        plsc.subcore_barrier()
        pltpu.sync_copy(acc.at[pl.ds(sid * wrt, wrt)],
                        out.at[cid].at[pl.ds(sid * wrt, wrt)])
        plsc.subcore_barrier()


def _sc_protein(projs, srcs2, dsts2):
    out_type = [jax.ShapeDtypeStruct((2, PR_NOUT, D), jnp.float32)
                for _ in range(3)]
    scratch = [
        pltpu.VMEM_SHARED((PR_NOUT, D), jnp.float32),
        pltpu.VMEM((NBUF, EB), jnp.int32),
        pltpu.VMEM((NBUF, EB), jnp.int32),
        pltpu.VMEM((NBUF, EB, D), jnp.float32),
        pltpu.VMEM((ZROWS, D), jnp.float32),
        pltpu.SemaphoreType.DMA,
        pltpu.SemaphoreType.DMA,
        pltpu.SemaphoreType.DMA,
    ]
    mesh = plsc.VectorSubcoreMesh(core_axis_name="c", subcore_axis_name="s")
    fn = pl.kernel(
        _sc_prot_body,
        out_type=out_type,
        mesh=mesh,
        scratch_types=scratch,
        compiler_params=pltpu.CompilerParams(use_tc_tiling_on_sc=False),
    )
    args = []
    for name in _PR_ETS:
        args += [projs[name], srcs2[name], dsts2[name]]
    res = fn(*args)
    return {name: res[i] for i, name in enumerate(_PR_ETS)}


def _pad_edges(edge, n_src, n_dst, npad, epad):
    e = edge.shape[1]
    extra = epad - e
    fill_src = jnp.arange(extra, dtype=jnp.int32) % n_src
    fill_dst = n_dst + jnp.arange(extra, dtype=jnp.int32) % (npad - n_dst)
    src = jnp.concatenate([edge[0], fill_src]).reshape(epad // EB, EB)
    dst = jnp.concatenate([edge[1], fill_dst]).reshape(epad // EB, EB)
    return src, dst


def kernel(gene_x, peak_x, protein_x, params, edge_g2p, edge_g2pr, edge_p2pr,
           edge_pr2pr):
    h = {"gene": gene_x, "peak": peak_x, "protein": protein_x}
    edges = {"g2p": edge_g2p, "g2pr": edge_g2pr, "p2pr": edge_p2pr,
             "pr2pr": edge_pr2pr}

    srcs2 = {}
    dsts2 = {}
    for name, n_src, n_dst, npad, epad in _PAD_INFO:
        srcs2[name], dsts2[name] = _pad_edges(edges[name], n_src, n_dst, npad,
                                              epad)

    cnts = None
    for l in range(NLAYERS):
        projs = {}
        selfs = {}
        for nt in NODE_TYPES:
            names = OUT_EDGES[nt]
            w_edges = [params["W_%s_%d" % (name, l)] for name in names]
            chunked = [name == "g2p" for name in names]
            outs = _project(h[nt], params["Wself_%s_%d" % (nt, l)], w_edges,
                            chunked)
            selfs[nt] = outs[0]
            for j, name in enumerate(names):
                projs[name] = outs[1 + j]

        cdsts = None
        if cnts is None:
            cdsts = [dsts2[name] for name in _PR_ETS]
        agg_g2p, new_cnts = _sc_g2p(projs["g2p"], srcs2["g2p"], dsts2["g2p"],
                                    cdsts)
        if new_cnts is not None:
            cnts = {"g2p": new_cnts[0], "g2pr": new_cnts[1],
                    "p2pr": new_cnts[2], "pr2pr": new_cnts[3]}
        aggs = _sc_protein(projs, srcs2, dsts2)
        aggs["g2p"] = agg_g2p

        new_h = {}
        for nt in NODE_TYPES:
            names = IN_EDGES[nt]
            new_h[nt] = _combine(selfs[nt], h[nt],
                                 [aggs[m] for m in names],
                                 [cnts[m] for m in names],
                                 [m == "g2p" for m in names])
        h = new_h

    outs = []
    for nt in NODE_TYPES:
        p = params
        outs.append(_encode(h[nt], p["enc_%s_W1" % nt], p["enc_%s_b1" % nt],
                            p["enc_%s_W2" % nt], p["enc_%s_b2" % nt],
                            p["enc_%s_W3" % nt], p["enc_%s_b3" % nt]))
    return tuple(outs)


# speedup vs baseline: 1.6598x; 1.0917x over previous
import functools
import jax
import jax.numpy as jnp
from jax import lax
from jax.experimental import pallas as pl
from jax.experimental.pallas import tpu as pltpu
from jax.experimental.pallas import tpu_sc as plsc

D = 256
NTOPIC = 20
NLAYERS = 3
NCHUNK = 8
CW = 32

NODE_TYPES = ("gene", "peak", "protein")
EDGE_DEFS = (("g2p", "gene", "peak"), ("g2pr", "gene", "protein"),
             ("p2pr", "peak", "protein"), ("pr2pr", "protein", "protein"))
IN_EDGES = {"gene": (), "peak": ("g2p",), "protein": ("g2pr", "p2pr", "pr2pr")}
OUT_EDGES = {"gene": ("g2p", "g2pr"), "peak": ("p2pr",), "protein": ("pr2pr",)}


def _row_block(n):
    if n % 1000 == 0:
        return 1000
    return n


def _proj_kernel(chunked, x_ref, w_ref, self_ref, *edge_refs):
    acc = jnp.dot(x_ref[...], w_ref[...], preferred_element_type=jnp.float32)
    self_ref[...] = acc[:, :D]
    for j, ch in enumerate(chunked):
        base = D * (1 + j)
        if ch:
            for c in range(NCHUNK):
                edge_refs[j][c] = acc[:, base + c * CW: base + (c + 1) * CW]
        else:
            edge_refs[j][...] = acc[:, base: base + D]


def _project(x, w_self, w_edges, chunked):
    n = x.shape[0]
    bn = _row_block(n)
    w = jnp.concatenate([w_self] + list(w_edges), axis=1)
    kout = w.shape[1]
    out_shapes = [jax.ShapeDtypeStruct((n, D), jnp.float32)]
    out_specs = [pl.BlockSpec((bn, D), lambda i: (i, 0))]
    for ch in chunked:
        if ch:
            out_shapes.append(jax.ShapeDtypeStruct((NCHUNK, n, CW), jnp.float32))
            out_specs.append(pl.BlockSpec((NCHUNK, bn, CW), lambda i: (0, i, 0)))
        else:
            out_shapes.append(jax.ShapeDtypeStruct((n, D), jnp.float32))
            out_specs.append(pl.BlockSpec((bn, D), lambda i: (i, 0)))
    return pl.pallas_call(
        functools.partial(_proj_kernel, tuple(chunked)),
        grid=(n // bn,),
        in_specs=[
            pl.BlockSpec((bn, D), lambda i: (i, 0)),
            pl.BlockSpec((D, kout), lambda i: (0, 0)),
        ],
        out_specs=out_specs,
        out_shape=out_shapes,
    )(x, w)


def _combine_kernel(chunked, self_ref, h_ref, *rest):
    out_ref = rest[-1]
    total = self_ref[...]
    for j, ch in enumerate(chunked):
        agg_ref = rest[2 * j]
        cnt_ref = rest[2 * j + 1]
        if ch:
            agg = jnp.concatenate([agg_ref[c] for c in range(NCHUNK)], axis=1)
        else:
            agg = agg_ref[0] + agg_ref[1]
        cnt = jnp.maximum(cnt_ref[:, 0:1], 1.0)
        total = total + agg / cnt
    out_ref[...] = jnp.maximum(total, 0.0) + h_ref[...]


def _combine(selfp, h, aggs, cnts, chunked):
    n = h.shape[0]
    bn = _row_block(n)
    in_specs = [pl.BlockSpec((bn, D), lambda i: (i, 0)),
                pl.BlockSpec((bn, D), lambda i: (i, 0))]
    args = [selfp, h]
    for agg, cnt, ch in zip(aggs, cnts, chunked):
        if ch:
            in_specs.append(pl.BlockSpec((NCHUNK, bn, CW), lambda i: (0, i, 0)))
        else:
            in_specs.append(pl.BlockSpec((2, bn, D), lambda i: (0, i, 0)))
        in_specs.append(pl.BlockSpec((bn, CW), lambda i: (i, 0)))
        args.append(agg)
        args.append(cnt)
    return pl.pallas_call(
        functools.partial(_combine_kernel, tuple(chunked)),
        grid=(n // bn,),
        in_specs=in_specs,
        out_specs=pl.BlockSpec((bn, D), lambda i: (i, 0)),
        out_shape=jax.ShapeDtypeStruct((n, D), jnp.float32),
    )(*args)


def _encoder_kernel(x_ref, w1_ref, b1_ref, w2_ref, b2_ref, w3_ref, b3_ref,
                    out_ref):
    h1 = jnp.maximum(
        jnp.dot(x_ref[...], w1_ref[...], preferred_element_type=jnp.float32)
        + b1_ref[...], 0.0)
    h2 = jnp.maximum(
        jnp.dot(h1, w2_ref[...], preferred_element_type=jnp.float32)
        + b2_ref[...], 0.0)
    mu = jnp.dot(h2, w3_ref[...], preferred_element_type=jnp.float32) + b3_ref[...]
    mu = mu - jnp.max(mu, axis=-1, keepdims=True)
    e = jnp.exp(mu)
    out_ref[...] = e / jnp.sum(e, axis=-1, keepdims=True)


def _encode(x, w1, b1, w2, b2, w3, b3):
    n = x.shape[0]
    bn = _row_block(n)
    h1 = w1.shape[1]
    h2 = w2.shape[1]
    return pl.pallas_call(
        _encoder_kernel,
        grid=(n // bn,),
        in_specs=[
            pl.BlockSpec((bn, D), lambda i: (i, 0)),
            pl.BlockSpec((D, h1), lambda i: (0, 0)),
            pl.BlockSpec((1, h1), lambda i: (0, 0)),
            pl.BlockSpec((h1, h2), lambda i: (0, 0)),
            pl.BlockSpec((1, h2), lambda i: (0, 0)),
            pl.BlockSpec((h2, NTOPIC), lambda i: (0, 0)),
            pl.BlockSpec((1, NTOPIC), lambda i: (0, 0)),
        ],
        out_specs=pl.BlockSpec((bn, NTOPIC), lambda i: (i, 0)),
        out_shape=jax.ShapeDtypeStruct((n, NTOPIC), jnp.float32),
    )(x, w1, b1.reshape(1, -1), w2, b2.reshape(1, -1), w3, b3.reshape(1, -1))


SC_NCORE = 2
SC_NSUB = 16
EB = 128
K_GRP = 2
NBUF = 2
ZROWS = 64

G2P_NSRC = 10000
G2P_NDST = 50000
G2P_NOUT = 50048
G2P_NPAD = 51200
G2P_EPAD = 163840
PR_NOUT = 2048
PR_EPAD = {"g2pr": 40960, "p2pr": 81920, "pr2pr": 24576}
_PAD_INFO = (
    ("g2p", G2P_NSRC, G2P_NDST, G2P_NPAD, G2P_EPAD),
    ("g2pr", 10000, 2000, PR_NOUT, PR_EPAD["g2pr"]),
    ("p2pr", 50000, 2000, PR_NOUT, PR_EPAD["p2pr"]),
    ("pr2pr", 2000, 2000, PR_NOUT, PR_EPAD["pr2pr"]),
)


def _init_const(buf, nrows, width, val):
    def body(i, carry):
        for j in range(width // 16):
            buf[i, pl.ds(16 * j, 16)] = jnp.full((16,), val, jnp.float32)
        return carry

    lax.fori_loop(0, nrows, body, 0)


def _sc_g2p_body(with_counts, *refs):
    if with_counts:
        (proj, sd2, sdc1, sdc2, sdc3,
         agg_out, cnt1, cnt2, cnt3, cnt4,
         acc, idx_sd, rows, zbuf, ones, gsem0, gsem1, ssem) = refs
    else:
        (proj, sd2, agg_out,
         acc, idx_sd, rows, zbuf, gsem0, gsem1, ssem) = refs
    gsems = (gsem0, gsem1)
    cid = lax.axis_index("c")
    sid = lax.axis_index("s")

    _init_const(zbuf, ZROWS, CW, 0.0)
    if with_counts:
        _init_const(ones, EB, CW, 1.0)

    def zero_acc(npad):
        rpt = npad // SC_NSUB

        def zb(i, carry):
            d0 = pltpu.async_copy(
                zbuf, acc.at[pl.ds(sid * rpt + 2 * i * ZROWS, ZROWS)], gsem0)
            d1 = pltpu.async_copy(
                zbuf, acc.at[pl.ds(sid * rpt + (2 * i + 1) * ZROWS, ZROWS)],
                gsem1)
            d0.wait()
            d1.wait()
            return carry

        lax.fori_loop(0, rpt // ZROWS // 2, zb, 0)

    nb = G2P_EPAD // SC_NSUB // EB
    ngrp = nb // K_GRP

    def load_idx(g, slot):
        r0 = sid * nb + g * K_GRP
        pltpu.sync_copy(sd2.at[pl.ds(r0, K_GRP)],
                        idx_sd.at[pl.ds(slot * K_GRP, K_GRP)])

    wrt = G2P_NOUT // SC_NSUB
    for c_l in range(NCHUNK // SC_NCORE):
        chunk = cid * (NCHUNK // SC_NCORE) + c_l
        zero_acc(G2P_NPAD)
        plsc.subcore_barrier()
        ptab = proj.at[chunk]

        def fire(bank, g):
            load_idx(g, bank)
            return [
                pltpu.async_copy(ptab.at[idx_sd.at[bank * K_GRP + k, 0]],
                                 rows.at[bank * K_GRP + k], gsems[bank])
                for k in range(K_GRP)
            ]

        def scatter(bank):
            sds = [
                pltpu.async_copy(rows.at[bank * K_GRP + k],
                                 acc.at[idx_sd.at[bank * K_GRP + k, 1]],
                                 ssem, add=True)
                for k in range(K_GRP)
            ]
            for d in sds:
                d.wait()

        def grp(i, carry):
            g0 = 2 * i
            da = fire(0, g0)
            db = fire(1, g0 + 1)
            for d in da:
                d.wait()
            scatter(0)
            for d in db:
                d.wait()
            scatter(1)
            return carry

        lax.fori_loop(0, ngrp // 2, grp, 0)
        plsc.subcore_barrier()
        pltpu.sync_copy(acc.at[pl.ds(sid * wrt, wrt)],
                        agg_out.at[chunk].at[pl.ds(sid * wrt, wrt)])
        plsc.subcore_barrier()

    if with_counts:
        cdefs = (
            (sd2, G2P_EPAD, 0, cnt1, G2P_NPAD, G2P_NOUT),
            (sdc1, PR_EPAD["g2pr"], 1, cnt2, PR_NOUT, PR_NOUT),
            (sdc2, PR_EPAD["p2pr"], 1, cnt3, PR_NOUT, PR_NOUT),
            (sdc3, PR_EPAD["pr2pr"], 1, cnt4, PR_NOUT, PR_NOUT),
        )
        for dref, epad, owner, cout, npad, nout in cdefs:
            cnb = epad // SC_NSUB // EB
            cngrp = cnb // K_GRP
            cwrt = nout // SC_NSUB

            @pl.when(cid == owner)
            def _():
                zero_acc(npad)
                plsc.subcore_barrier()

                def load_dst(g, slot):
                    r0 = sid * cnb + g * K_GRP
                    pltpu.sync_copy(dref.at[pl.ds(r0, K_GRP)],
                                    idx_sd.at[pl.ds(slot * K_GRP, K_GRP)])

                def cscatter(bank):
                    sds = [
                        pltpu.async_copy(
                            ones, acc.at[idx_sd.at[bank * K_GRP + k, 1]],
                            ssem, add=True)
                        for k in range(K_GRP)
                    ]
                    for d in sds:
                        d.wait()

                def cgrp(i, carry):
                    g0 = 2 * i
                    load_dst(g0, 0)
                    load_dst(g0 + 1, 1)
                    cscatter(0)
                    cscatter(1)
                    return carry

                lax.fori_loop(0, cngrp // 2, cgrp, 0)
                plsc.subcore_barrier()
                pltpu.sync_copy(acc.at[pl.ds(sid * cwrt, cwrt)],
                                cout.at[pl.ds(sid * cwrt, cwrt)])
                plsc.subcore_barrier()


def _sc_g2p(proj, sd2, cdsts):
    with_counts = cdsts is not None
    out_type = [jax.ShapeDtypeStruct((NCHUNK, G2P_NOUT, CW), jnp.float32)]
    if with_counts:
        out_type += [jax.ShapeDtypeStruct((G2P_NOUT, CW), jnp.float32)] + [
            jax.ShapeDtypeStruct((PR_NOUT, CW), jnp.float32) for _ in range(3)]
    scratch = [
        pltpu.VMEM_SHARED((G2P_NPAD, CW), jnp.float32),
        pltpu.VMEM((NBUF * K_GRP, 2, EB), jnp.int32),
        pltpu.VMEM((NBUF * K_GRP, EB, CW), jnp.float32),
        pltpu.VMEM((ZROWS, CW), jnp.float32),
    ]
    if with_counts:
        scratch.append(pltpu.VMEM((EB, CW), jnp.float32))
    scratch += [pltpu.SemaphoreType.DMA] * 3
    mesh = plsc.VectorSubcoreMesh(core_axis_name="c", subcore_axis_name="s")
    fn = pl.kernel(
        functools.partial(_sc_g2p_body, with_counts),
        out_type=out_type,
        mesh=mesh,
        scratch_types=scratch,
        compiler_params=pltpu.CompilerParams(use_tc_tiling_on_sc=False),
    )
    args = [proj, sd2] + (list(cdsts) if with_counts else [])
    res = fn(*args)
    if with_counts:
        return res[0], list(res[1:5])
    return res[0], None


_PR_ETS = ("g2pr", "p2pr", "pr2pr")


def _sc_prot_body(*refs):
    (p1, sd1, p2, sd2, p3, sd3, o1, o2, o3,
     acc, idx_sd, rows, zbuf, gsem0, gsem1, ssem) = refs
    gsems = (gsem0, gsem1)
    cid = lax.axis_index("c")
    sid = lax.axis_index("s")
    wid = cid * SC_NSUB + sid

    _init_const(zbuf, ZROWS, D, 0.0)

    rpt = PR_NOUT // SC_NSUB

    def zero_acc():
        d0 = pltpu.async_copy(zbuf, acc.at[pl.ds(sid * rpt, ZROWS)], gsem0)
        d1 = pltpu.async_copy(zbuf, acc.at[pl.ds(sid * rpt + ZROWS, ZROWS)],
                              gsem1)
        d0.wait()
        d1.wait()

    wrt = PR_NOUT // SC_NSUB
    for proj, sdx, out, name in ((p1, sd1, o1, "g2pr"),
                                 (p2, sd2, o2, "p2pr"),
                                 (p3, sd3, o3, "pr2pr")):
        epad = PR_EPAD[name]
        nb = epad // (2 * SC_NSUB) // EB

        def load_idx(g, slot):
            r0 = wid * nb + g
            pltpu.sync_copy(sdx.at[pl.ds(r0, 1)], idx_sd.at[pl.ds(slot, 1)])

        zero_acc()
        plsc.subcore_barrier()

        def fire(bank, g):
            load_idx(g, bank)
            return pltpu.async_copy(proj.at[idx_sd.at[bank, 0]],
                                    rows.at[bank], gsems[bank])

        def scatter(bank):
            pltpu.async_copy(rows.at[bank], acc.at[idx_sd.at[bank, 1]], ssem,
                             add=True).wait()

        def grp(i, carry):
            g0 = 2 * i
            da = fire(0, g0)
            db = fire(1, g0 + 1)
            da.wait()
            scatter(0)
            db.wait()
            scatter(1)
            return carry

        lax.fori_loop(0, nb // 2, grp, 0)
        plsc.subcore_barrier()
        pltpu.sync_copy(acc.at[pl.ds(sid * wrt, wrt)],
                        out.at[cid].at[pl.ds(sid * wrt, wrt)])
        plsc.subcore_barrier()


def _sc_protein(projs, eidx):
    out_type = [jax.ShapeDtypeStruct((2, PR_NOUT, D), jnp.float32)
                for _ in range(3)]
    scratch = [
        pltpu.VMEM_SHARED((PR_NOUT, D), jnp.float32),
        pltpu.VMEM((NBUF, 2, EB), jnp.int32),
        pltpu.VMEM((NBUF, EB, D), jnp.float32),
        pltpu.VMEM((ZROWS, D), jnp.float32),
        pltpu.SemaphoreType.DMA,
        pltpu.SemaphoreType.DMA,
        pltpu.SemaphoreType.DMA,
    ]
    mesh = plsc.VectorSubcoreMesh(core_axis_name="c", subcore_axis_name="s")
    fn = pl.kernel(
        _sc_prot_body,
        out_type=out_type,
        mesh=mesh,
        scratch_types=scratch,
        compiler_params=pltpu.CompilerParams(use_tc_tiling_on_sc=False),
    )
    args = []
    for name in _PR_ETS:
        args += [projs[name], eidx[name]]
    res = fn(*args)
    return {name: res[i] for i, name in enumerate(_PR_ETS)}


def _pad_edges(edge, n_src, n_dst, npad, epad):
    e = edge.shape[1]
    extra = epad - e
    fill_src = jnp.arange(extra, dtype=jnp.int32) % n_src
    fill_dst = n_dst + jnp.arange(extra, dtype=jnp.int32) % (npad - n_dst)
    src = jnp.concatenate([edge[0], fill_src]).reshape(epad // EB, EB)
    dst = jnp.concatenate([edge[1], fill_dst]).reshape(epad // EB, EB)
    return jnp.stack([src, dst], axis=1)


def kernel(gene_x, peak_x, protein_x, params, edge_g2p, edge_g2pr, edge_p2pr,
           edge_pr2pr):
    h = {"gene": gene_x, "peak": peak_x, "protein": protein_x}
    edges = {"g2p": edge_g2p, "g2pr": edge_g2pr, "p2pr": edge_p2pr,
             "pr2pr": edge_pr2pr}

    eidx = {}
    for name, n_src, n_dst, npad, epad in _PAD_INFO:
        eidx[name] = _pad_edges(edges[name], n_src, n_dst, npad, epad)

    cnts = None
    for l in range(NLAYERS):
        projs = {}
        selfs = {}
        for nt in NODE_TYPES:
            names = OUT_EDGES[nt]
            w_edges = [params["W_%s_%d" % (name, l)] for name in names]
            chunked = [name == "g2p" for name in names]
            outs = _project(h[nt], params["Wself_%s_%d" % (nt, l)], w_edges,
                            chunked)
            selfs[nt] = outs[0]
            for j, name in enumerate(names):
                projs[name] = outs[1 + j]

        cdsts = None
        if cnts is None:
            cdsts = [eidx[name] for name in _PR_ETS]
        agg_g2p, new_cnts = _sc_g2p(projs["g2p"], eidx["g2p"], cdsts)
        if new_cnts is not None:
            cnts = {"g2p": new_cnts[0], "g2pr": new_cnts[1],
                    "p2pr": new_cnts[2], "pr2pr": new_cnts[3]}
        aggs = _sc_protein(projs, eidx)
        aggs["g2p"] = agg_g2p

        new_h = {}
        for nt in NODE_TYPES:
            names = IN_EDGES[nt]
            new_h[nt] = _combine(selfs[nt], h[nt],
                                 [aggs[m] for m in names],
                                 [cnts[m] for m in names],
                                 [m == "g2p" for m in names])
        h = new_h

    outs = []
    for nt in NODE_TYPES:
        p = params
        outs.append(_encode(h[nt], p["enc_%s_W1" % nt], p["enc_%s_b1" % nt],
                            p["enc_%s_W2" % nt], p["enc_%s_b2" % nt],
                            p["enc_%s_W3" % nt], p["enc_%s_b3" % nt]))
    return tuple(outs)
